# baseline (speedup 1.0000x reference)
import functools

import jax
import jax.numpy as jnp
from jax import lax
from jax.experimental import pallas as pl
from jax.experimental.pallas import tpu as pltpu
from jax.experimental.pallas import tpu_sc as plsc

S = 16384
HALF = S // 2
FEAT = 128
K = 512
BLK = 4096
OB = 1024


def _sc_gather(table, idx, start, n, tidx=None):
    info = plsc.get_sparse_core_info()
    nc, ns = info.num_cores, info.num_subcores
    nw = nc * ns
    bpw = n // nw
    d = table.shape[1]
    mesh = plsc.VectorSubcoreMesh(core_axis_name="c", subcore_axis_name="s")

    out_type = [jax.ShapeDtypeStruct((n, d), table.dtype)]
    scratch = [
        pltpu.VMEM((bpw,), jnp.int32),
        pltpu.VMEM((bpw, d), table.dtype),
        pltpu.SemaphoreType.DMA,
    ]
    if tidx is not None:
        out_type.append(jax.ShapeDtypeStruct((8, d), table.dtype))
        scratch += [pltpu.VMEM((8,), jnp.int32), pltpu.VMEM((8, d), table.dtype),
                    pltpu.SemaphoreType.DMA]

    @functools.partial(
        pl.kernel, mesh=mesh, out_type=tuple(out_type), scratch_types=scratch,
    )
    def gk(table_hbm, idx_hbm, *rest):
        if tidx is not None:
            (tidx_hbm, out_hbm, out2_hbm, idx_v, rows_v, sem,
             tidx_v, rows2_v, sem2) = rest
        else:
            out_hbm, idx_v, rows_v, sem = rest
        wid = lax.axis_index("s") * nc + lax.axis_index("c")
        base = start + wid * bpw
        pltpu.sync_copy(idx_hbm.at[pl.ds(base, bpw)], idx_v)
        copies = []
        off = 0
        while off < bpw:
            sz = min(128, bpw - off)
            copies.append(
                pltpu.async_copy(
                    table_hbm.at[idx_v.at[pl.ds(off, sz)]],
                    rows_v.at[pl.ds(off, sz)],
                    sem,
                )
            )
            off += sz
        if tidx is not None:
            @pl.when(wid == 0)
            def _():
                pltpu.sync_copy(tidx_hbm, tidx_v)
                pltpu.async_copy(table_hbm.at[tidx_v], rows2_v, sem2).wait()
                pltpu.sync_copy(rows2_v, out2_hbm)
        for c in copies:
            c.wait()
        pltpu.sync_copy(rows_v, out_hbm.at[pl.ds(wid * bpw, bpw)])

    if tidx is not None:
        return gk(table, idx, tidx)
    return gk(table, idx)[0]


_DN = (((1,), (1,)), ((), ()))


def _mlp_kernel(g, adj, w1, w2, l1wt, l1b, l2wt, l2b, l3w, l3b,
                tar, add, wlab, wsec, out_ref, ta_scr):
    i = pl.program_id(0)

    @pl.when(i == 0)
    def _():
        ta_scr[:, 0:16] = jnp.dot(jnp.dot(tar[0:1, :], w1[...]), w2[...])
        ta_scr[:, 16:32] = jnp.dot(jnp.dot(add[...], w1[...]), w2[...])

    sub_xw = jnp.dot(jnp.dot(g[...], w1[...]), w2[...])
    concat = jnp.concatenate([
        jnp.broadcast_to(ta_scr[:, 0:16], (BLK, 16)),
        sub_xw,
        jnp.broadcast_to(ta_scr[:, 16:32], (BLK, 16)),
        adj[...],
        jnp.broadcast_to(wlab[...], (BLK, FEAT)),
        jnp.broadcast_to(wsec[...], (BLK, FEAT)),
        jnp.zeros((BLK, 79), jnp.float32),
    ], axis=1)
    h = jnp.dot(concat, l1wt[...]) + l1b[...]
    h = jnp.where(h >= 0, h, 0.01 * h)
    h = jnp.dot(h, l2wt[...]) + l2b[...]
    h = jnp.where(h >= 0, h, 0.01 * h)
    ht = lax.transpose(h, (1, 0))
    for j in range(BLK // OB):
        o = jnp.dot(l3w[...], ht[:, j * OB:(j + 1) * OB]) + l3b[...]
        out_ref[j:j + 1, 0, :] = o


def _mlp_call(g, adj, row0, w1, w2, l1wt, l1b, l2wt, l2b, l3w, l3b,
              tar, add, wlab, wsec, interpret=False):
    n = g.shape[0]
    grid = (n // BLK,)
    blk0 = row0 // BLK

    def full(x):
        return pl.BlockSpec(x.shape, lambda i: (0,) * x.ndim)

    in_specs = [
        pl.BlockSpec((BLK, FEAT), lambda i: (i, 0)),
        pl.BlockSpec((BLK, 1), lambda i: (i + blk0, 0)),
    ] + [full(x) for x in (w1, w2, l1wt, l1b, l2wt, l2b, l3w, l3b,
                           tar, add, wlab, wsec)]
    rpb = BLK // OB
    return pl.pallas_call(
        _mlp_kernel,
        grid=grid,
        in_specs=in_specs,
        out_specs=pl.BlockSpec((rpb, 1, OB), lambda i: (i, 0, 0)),
        out_shape=jax.ShapeDtypeStruct((n // OB, 1, OB), jnp.float32),
        scratch_shapes=[pltpu.VMEM((1, 32), jnp.float32)],
        interpret=interpret,
    )(g, adj, w1, w2, l1wt, l1b, l2wt, l2b, l3w, l3b, tar, add, wlab, wsec)


def _linear_prefix(eqi):
    rows, cols = eqi.shape
    lane = lax.broadcasted_iota(jnp.int32, eqi.shape, 1)
    rowc = eqi
    k = 1
    while k < cols:
        sh = pltpu.roll(rowc, k, 1)
        rowc = rowc + jnp.where(lane >= k, sh, 0)
        k *= 2
    rowtot = rowc[:, cols - 1:cols]
    sub = lax.broadcasted_iota(jnp.int32, (rows, 1), 0)
    off = rowtot
    k = 1
    while k < rows:
        sh = pltpu.roll(off, k, 0)
        off = off + jnp.where(sub >= k, sh, 0)
        k *= 2
    rowoff = off - rowtot
    return rowc + rowoff


def _sum11(x):
    return jnp.sum(jnp.sum(x, axis=1, keepdims=True), axis=0, keepdims=True)


def _select_kernel(l0_ref, l1_ref, ov_ref, out_ref):
    rows = HALF // OB
    a = l0_ref[...].reshape(rows, OB)
    b = l1_ref[...].reshape(rows, OB)
    m = jnp.maximum(jnp.max(a), jnp.max(b))
    ea = jnp.exp(a - m)
    eb = jnp.exp(b - m)
    s = jnp.sum(ea) + jnp.sum(eb)
    sa = ea / s
    sb = eb / s
    ba = lax.bitcast_convert_type(sa, jnp.int32)
    bb = lax.bitcast_convert_type(sb, jnp.int32)

    def tbody(_, lohi):
        lo, hi = lohi
        mid = lo + (hi - lo) // 2
        cnt = (_sum11(jnp.where(ba > mid, 1, 0))
               + _sum11(jnp.where(bb > mid, 1, 0)))
        below = cnt < K
        return (jnp.where(below, lo, mid), jnp.where(below, mid, hi))

    lo0 = jnp.full((1, 1), -1, jnp.int32)
    hi0 = jnp.full((1, 1), 0x7F800000, jnp.int32)
    _, t = lax.fori_loop(0, 32, tbody, (lo0, hi0))
    gta = ba > t
    gtb = bb > t
    eqa = (ba == t).astype(jnp.int32)
    eqb = (bb == t).astype(jnp.int32)
    need = (K - _sum11(jnp.where(gta, 1, 0)) - _sum11(jnp.where(gtb, 1, 0)))
    inca = _linear_prefix(eqa)
    incb = _linear_prefix(eqb) + _sum11(eqa)
    ov = ov_ref[0, 0]
    zero = jnp.float32(0.0)
    va = jnp.where(gta | ((eqa > 0) & (inca <= need)), ov, zero)
    vb = jnp.where(gtb | ((eqb > 0) & (incb <= need)), ov, zero)
    out_ref[...] = jnp.concatenate([va, vb], axis=0)


def _select_call(l0, l1, one_val, interpret=False):
    def full(x):
        return pl.BlockSpec(x.shape, lambda: (0,) * x.ndim)

    return pl.pallas_call(
        _select_kernel,
        in_specs=[full(l0), full(l1),
                  pl.BlockSpec(memory_space=pltpu.SMEM)],
        out_specs=pl.BlockSpec((S // OB, OB), lambda: (0, 0)),
        out_shape=jax.ShapeDtypeStruct((S // OB, OB), jnp.float32),
        interpret=interpret,
    )(l0, l1, one_val)


def kernel(budget, target, sub_graph_nodes, new_feat, adj_tensor, wlabel, wsec,
           weight1, weight2, l1_w, l1_b, l2_w, l2_b, l3_w, l3_b):
    idx = sub_graph_nodes.astype(jnp.int32)
    tidx = jnp.concatenate([target.astype(jnp.int32), jnp.zeros((7,), jnp.int32)])
    g0, tar_rows = _sc_gather(new_feat, idx, 0, HALF, tidx=tidx)
    g1 = _sc_gather(new_feat, idx, HALF, HALF)

    l1wt = jnp.concatenate([l1_w.T, jnp.zeros((79, 512), l1_w.dtype)], axis=0)
    l2wt = l2_w.T
    l1b = l1_b.reshape(1, 512)
    l2b = l2_b.reshape(1, 32)
    l3b = l3_b.reshape(1, 1)
    add_row = new_feat[-1:]

    lg0 = _mlp_call(g0, adj_tensor, 0, weight1, weight2, l1wt, l1b, l2wt,
                    l2b, l3_w, l3b, tar_rows, add_row, wlabel, wsec)
    lg1 = _mlp_call(g1, adj_tensor, HALF, weight1, weight2, l1wt, l1b, l2wt,
                    l2b, l3_w, l3b, tar_rows, add_row, wlabel, wsec)

    one_val = (jnp.asarray(budget, jnp.float32) / jnp.float32(K)).reshape(1, 1)
    hard = _select_call(lg0, lg1, one_val)
    score = hard.reshape(S)
    score_idx = sub_graph_nodes.reshape(1, -1)
    return (score, score_idx)


# speedup vs baseline: 1.0314x; 1.0314x over previous
import functools

import jax
import jax.numpy as jnp
from jax import lax
from jax.experimental import pallas as pl
from jax.experimental.pallas import tpu as pltpu
from jax.experimental.pallas import tpu_sc as plsc

S = 16384
HALF = S // 2
FEAT = 128
K = 512
BLK = 4096
OB = 1024


def _sc_gather(table, idx, start, n, tidx=None):
    info = plsc.get_sparse_core_info()
    nc, ns = info.num_cores, info.num_subcores
    nw = nc * ns
    bpw = n // nw
    d = table.shape[1]
    mesh = plsc.VectorSubcoreMesh(core_axis_name="c", subcore_axis_name="s")

    out_type = [jax.ShapeDtypeStruct((n, d), table.dtype)]
    scratch = [
        pltpu.VMEM((bpw,), jnp.int32),
        pltpu.VMEM((bpw, d), table.dtype),
        pltpu.SemaphoreType.DMA,
    ]
    if tidx is not None:
        out_type.append(jax.ShapeDtypeStruct((8, d), table.dtype))
        scratch += [pltpu.VMEM((8,), jnp.int32), pltpu.VMEM((8, d), table.dtype),
                    pltpu.SemaphoreType.DMA]

    @functools.partial(
        pl.kernel, mesh=mesh, out_type=tuple(out_type), scratch_types=scratch,
    )
    def gk(table_hbm, idx_hbm, *rest):
        if tidx is not None:
            (tidx_hbm, out_hbm, out2_hbm, idx_v, rows_v, sem,
             tidx_v, rows2_v, sem2) = rest
        else:
            out_hbm, idx_v, rows_v, sem = rest
        wid = lax.axis_index("s") * nc + lax.axis_index("c")
        base = start + wid * bpw
        pltpu.sync_copy(idx_hbm.at[pl.ds(base, bpw)], idx_v)
        copies = []
        off = 0
        while off < bpw:
            sz = min(128, bpw - off)
            copies.append(
                pltpu.async_copy(
                    table_hbm.at[idx_v.at[pl.ds(off, sz)]],
                    rows_v.at[pl.ds(off, sz)],
                    sem,
                )
            )
            off += sz
        if tidx is not None:
            @pl.when(wid == 0)
            def _():
                pltpu.sync_copy(tidx_hbm, tidx_v)
                pltpu.async_copy(table_hbm.at[tidx_v], rows2_v, sem2).wait()
                pltpu.sync_copy(rows2_v, out2_hbm)
        for c in copies:
            c.wait()
        pltpu.sync_copy(rows_v, out_hbm.at[pl.ds(wid * bpw, bpw)])

    if tidx is not None:
        return gk(table, idx, tidx)
    return gk(table, idx)[0]


_DN = (((1,), (1,)), ((), ()))


def _mlp_kernel(g, adj, w1, w2, l1wt, l1b, l2wt, l2b, l3w, l3b,
                tar, add, wlab, wsec, out_ref, ta_scr):
    i = pl.program_id(0)

    @pl.when(i == 0)
    def _():
        ta_scr[:, 0:16] = jnp.dot(jnp.dot(tar[0:1, :], w1[...]), w2[...])
        ta_scr[:, 16:32] = jnp.dot(jnp.dot(add[...], w1[...]), w2[...])

    sub_xw = jnp.dot(jnp.dot(g[...], w1[...]), w2[...])
    concat = jnp.concatenate([
        jnp.broadcast_to(ta_scr[:, 0:16], (BLK, 16)),
        sub_xw,
        jnp.broadcast_to(ta_scr[:, 16:32], (BLK, 16)),
        adj[...],
        jnp.broadcast_to(wlab[...], (BLK, FEAT)),
        jnp.broadcast_to(wsec[...], (BLK, FEAT)),
    ], axis=1)
    h = jnp.dot(concat, l1wt[...]) + l1b[...]
    h = jnp.where(h >= 0, h, 0.01 * h)
    h = jnp.dot(h, l2wt[...]) + l2b[...]
    h = jnp.where(h >= 0, h, 0.01 * h)
    ht = lax.transpose(h, (1, 0))
    for j in range(BLK // OB):
        o = jnp.dot(l3w[...], ht[:, j * OB:(j + 1) * OB]) + l3b[...]
        out_ref[j:j + 1, 0, :] = o


def _mlp_call(g, adj, row0, w1, w2, l1wt, l1b, l2wt, l2b, l3w, l3b,
              tar, add, wlab, wsec, interpret=False):
    n = g.shape[0]
    grid = (n // BLK,)
    blk0 = row0 // BLK

    def full(x):
        return pl.BlockSpec(x.shape, lambda i: (0,) * x.ndim)

    in_specs = [
        pl.BlockSpec((BLK, FEAT), lambda i: (i, 0)),
        pl.BlockSpec((BLK, 1), lambda i: (i + blk0, 0)),
    ] + [full(x) for x in (w1, w2, l1wt, l1b, l2wt, l2b, l3w, l3b,
                           tar, add, wlab, wsec)]
    rpb = BLK // OB
    return pl.pallas_call(
        _mlp_kernel,
        grid=grid,
        in_specs=in_specs,
        out_specs=pl.BlockSpec((rpb, 1, OB), lambda i: (i, 0, 0)),
        out_shape=jax.ShapeDtypeStruct((n // OB, 1, OB), jnp.float32),
        scratch_shapes=[pltpu.VMEM((1, 32), jnp.float32)],
        interpret=interpret,
    )(g, adj, w1, w2, l1wt, l1b, l2wt, l2b, l3w, l3b, tar, add, wlab, wsec)


def _linear_prefix(eqi):
    rows, cols = eqi.shape
    lane = lax.broadcasted_iota(jnp.int32, eqi.shape, 1)
    rowc = eqi
    k = 1
    while k < cols:
        sh = pltpu.roll(rowc, k, 1)
        rowc = rowc + jnp.where(lane >= k, sh, 0)
        k *= 2
    rowtot = rowc[:, cols - 1:cols]
    sub = lax.broadcasted_iota(jnp.int32, (rows, 1), 0)
    off = rowtot
    k = 1
    while k < rows:
        sh = pltpu.roll(off, k, 0)
        off = off + jnp.where(sub >= k, sh, 0)
        k *= 2
    rowoff = off - rowtot
    return rowc + rowoff


def _sum11(x):
    return jnp.sum(jnp.sum(x, axis=1, keepdims=True), axis=0, keepdims=True)


def _select_kernel(l0_ref, l1_ref, ov_ref, out_ref):
    rows = HALF // OB
    a = l0_ref[...].reshape(rows, OB)
    b = l1_ref[...].reshape(rows, OB)
    m = jnp.maximum(jnp.max(a), jnp.max(b))
    ea = jnp.exp(a - m)
    eb = jnp.exp(b - m)
    s = jnp.sum(ea) + jnp.sum(eb)
    sa = ea / s
    sb = eb / s
    ba = lax.bitcast_convert_type(sa, jnp.int32)
    bb = lax.bitcast_convert_type(sb, jnp.int32)

    def tbody(_, lohi):
        lo, hi = lohi
        mid = lo + (hi - lo) // 2
        cnt = (_sum11(jnp.where(ba > mid, 1, 0))
               + _sum11(jnp.where(bb > mid, 1, 0)))
        below = cnt < K
        return (jnp.where(below, lo, mid), jnp.where(below, mid, hi))

    lo0 = jnp.full((1, 1), -1, jnp.int32)
    hi0 = jnp.full((1, 1), 0x7F800000, jnp.int32)
    _, t = lax.fori_loop(0, 32, tbody, (lo0, hi0))
    gta = ba > t
    gtb = bb > t
    eqa = (ba == t).astype(jnp.int32)
    eqb = (bb == t).astype(jnp.int32)
    need = (K - _sum11(jnp.where(gta, 1, 0)) - _sum11(jnp.where(gtb, 1, 0)))
    inca = _linear_prefix(eqa)
    incb = _linear_prefix(eqb) + _sum11(eqa)
    ov = ov_ref[0, 0]
    zero = jnp.float32(0.0)
    va = jnp.where(gta | ((eqa > 0) & (inca <= need)), ov, zero)
    vb = jnp.where(gtb | ((eqb > 0) & (incb <= need)), ov, zero)
    out_ref[...] = jnp.concatenate([va, vb], axis=0)


def _select_call(l0, l1, one_val, interpret=False):
    def full(x):
        return pl.BlockSpec(x.shape, lambda: (0,) * x.ndim)

    return pl.pallas_call(
        _select_kernel,
        in_specs=[full(l0), full(l1),
                  pl.BlockSpec(memory_space=pltpu.SMEM)],
        out_specs=pl.BlockSpec((S // OB, OB), lambda: (0, 0)),
        out_shape=jax.ShapeDtypeStruct((S // OB, OB), jnp.float32),
        interpret=interpret,
    )(l0, l1, one_val)


def kernel(budget, target, sub_graph_nodes, new_feat, adj_tensor, wlabel, wsec,
           weight1, weight2, l1_w, l1_b, l2_w, l2_b, l3_w, l3_b):
    idx = sub_graph_nodes.astype(jnp.int32)
    tidx = jnp.concatenate([target.astype(jnp.int32), jnp.zeros((7,), jnp.int32)])
    g0, tar_rows = _sc_gather(new_feat, idx, 0, HALF, tidx=tidx)
    g1 = _sc_gather(new_feat, idx, HALF, HALF)

    l1wt = l1_w.T
    l2wt = l2_w.T
    l1b = l1_b.reshape(1, 512)
    l2b = l2_b.reshape(1, 32)
    l3b = l3_b.reshape(1, 1)
    add_row = new_feat[-1:]

    lg0 = _mlp_call(g0, adj_tensor, 0, weight1, weight2, l1wt, l1b, l2wt,
                    l2b, l3_w, l3b, tar_rows, add_row, wlabel, wsec)
    lg1 = _mlp_call(g1, adj_tensor, HALF, weight1, weight2, l1wt, l1b, l2wt,
                    l2b, l3_w, l3b, tar_rows, add_row, wlabel, wsec)

    one_val = (jnp.asarray(budget, jnp.float32) / jnp.float32(K)).reshape(1, 1)
    hard = _select_call(lg0, lg1, one_val)
    score = hard.reshape(S)
    score_idx = sub_graph_nodes.reshape(1, -1)
    return (score, score_idx)
